# baseline (speedup 1.0000x reference)
import functools

import jax
import jax.numpy as jnp
from jax import lax
from jax.experimental import pallas as pl
from jax.experimental.pallas import tpu as pltpu
from jax.experimental.pallas import tpu_sc as plsc

B, H, W, C = 4, 112, 112, 192
HW = H * W
HO, WO = 2 * H, 2 * W
P = HO * WO
NCORE, NSUB = 2, 16
NWORK = NCORE * NSUB
MPERW = C // NWORK
HB = 16
YB = 16
INV_C = 1.0 / C


def _group_body(x_ref, i_ref, vo_hbm, po_hbm, vscr, pscr):
    i = pl.program_id(0)
    v = x_ref[0]
    idx = i_ref[0]
    p0 = (idx.astype(jnp.float32) * INV_C).astype(jnp.int32)
    r = idx - p0 * C
    p = p0 + (r >= C).astype(jnp.int32) - (r < 0).astype(jnp.int32)
    vscr[...] = jnp.swapaxes(v, 0, 1)
    pscr[...] = jnp.swapaxes(p, 0, 1)
    pltpu.sync_copy(vscr, vo_hbm.at[:, pl.ds(i * HB, HB), :])
    pltpu.sync_copy(pscr, po_hbm.at[:, pl.ds(i * HB, HB), :])


def _group_channels(x, ii, b):
    return pl.pallas_call(
        _group_body,
        grid=(H // HB,),
        in_specs=[
            pl.BlockSpec((1, HB, C, W), lambda i: (b, i, 0, 0)),
            pl.BlockSpec((1, HB, C, W), lambda i: (b, i, 0, 0)),
        ],
        out_specs=[
            pl.BlockSpec(memory_space=pl.ANY),
            pl.BlockSpec(memory_space=pl.ANY),
        ],
        out_shape=[
            jax.ShapeDtypeStruct((C, H, W), jnp.float32),
            jax.ShapeDtypeStruct((C, H, W), jnp.int32),
        ],
        scratch_shapes=[
            pltpu.VMEM((C, HB, W), jnp.float32),
            pltpu.VMEM((C, HB, W), jnp.int32),
        ],
    )(x, ii)


def _relayout_first_body(t_hbm, o_ref, scr):
    y = pl.program_id(0)
    pltpu.sync_copy(t_hbm.at[:, pl.ds(y * YB, YB), :], scr)
    o_ref[0] = jnp.swapaxes(scr[...], 0, 1)


def _relayout_accum_body(prev_ref, t_hbm, o_ref, scr):
    del prev_ref
    y = pl.program_id(0)
    pltpu.sync_copy(t_hbm.at[:, pl.ds(y * YB, YB), :], scr)
    o_ref[0] = jnp.swapaxes(scr[...], 0, 1)


def _relayout_out(out_t, b, prev):
    scratch = [pltpu.VMEM((C, YB, WO), jnp.float32)]
    if prev is None:
        return pl.pallas_call(
            _relayout_first_body,
            grid=(HO // YB,),
            in_specs=[pl.BlockSpec(memory_space=pl.ANY)],
            out_specs=pl.BlockSpec((1, YB, C, WO), lambda y: (b, y, 0, 0)),
            out_shape=jax.ShapeDtypeStruct((B, HO, C, WO), jnp.float32),
            scratch_shapes=scratch,
        )(out_t)
    return pl.pallas_call(
        _relayout_accum_body,
        grid=(HO // YB,),
        in_specs=[
            pl.BlockSpec(memory_space=pl.ANY),
            pl.BlockSpec(memory_space=pl.ANY),
        ],
        out_specs=pl.BlockSpec((1, YB, C, WO), lambda y: (b, y, 0, 0)),
        out_shape=jax.ShapeDtypeStruct((B, HO, C, WO), jnp.float32),
        input_output_aliases={0: 0},
        scratch_shapes=scratch,
    )(prev, out_t)


_mesh = plsc.VectorSubcoreMesh(core_axis_name="c", subcore_axis_name="s")


@functools.partial(
    pl.kernel,
    out_type=jax.ShapeDtypeStruct((C, P), jnp.float32),
    mesh=_mesh,
    scratch_types=[
        pltpu.VMEM((P,), jnp.float32),
        pltpu.VMEM((2, H, W), jnp.float32),
        pltpu.VMEM((2, H, W), jnp.int32),
        pltpu.SemaphoreType.DMA((2,)),
    ],
    compiler_params=pltpu.CompilerParams(needs_layout_passes=False),
)
def _scatter(vals_hbm, p_hbm, out_hbm, canvas, vbuf, pbuf, lsem):
    w = lax.axis_index("s") * NCORE + lax.axis_index("c")
    zero16 = jnp.zeros((16,), jnp.float32)

    def _load(m, sel):
        c = w + NWORK * m
        pltpu.async_copy(vals_hbm.at[c], vbuf.at[sel], lsem.at[sel])
        pltpu.async_copy(p_hbm.at[c], pbuf.at[sel], lsem.at[sel])

    _load(jnp.int32(0), jnp.int32(0))

    def _zero(i, carry):
        canvas[pl.ds(i * 16, 16)] = zero16
        return carry

    lax.fori_loop(0, P // 16, _zero, 0)

    def _plane(m, carry):
        sel = jnp.bitwise_and(m, 1)

        @pl.when(m < MPERW - 1)
        def _():
            _load(m + 1, 1 - sel)

        c = w + NWORK * m
        pltpu.make_async_copy(vals_hbm.at[c], vbuf.at[sel], lsem.at[sel]).wait()
        pltpu.make_async_copy(p_hbm.at[c], pbuf.at[sel], lsem.at[sel]).wait()

        def _scat(h, carry3):
            for u in range(W // 16):
                idxv = pbuf[sel, h, pl.ds(u * 16, 16)]
                valv = vbuf[sel, h, pl.ds(u * 16, 16)]
                plsc.addupdate_scatter(canvas, [idxv], valv)
            return carry3

        lax.fori_loop(0, H, _scat, 0)
        pltpu.sync_copy(canvas, out_hbm.at[c])

        def _clear(h, carry3):
            for u in range(W // 16):
                idxv = pbuf[sel, h, pl.ds(u * 16, 16)]
                plsc.store_scatter(canvas, [idxv], zero16)
            return carry3

        lax.fori_loop(0, H, _clear, 0)
        return carry

    lax.fori_loop(0, MPERW, _plane, 0)


def kernel(input, indices):
    x = jnp.transpose(input, (0, 1, 3, 2))
    ii = jnp.transpose(indices.astype(jnp.int32), (0, 1, 3, 2))
    out = None
    for b in range(B):
        vt, pt = _group_channels(x, ii, b)
        out_t = _scatter(vt, pt)
        out = _relayout_out(out_t.reshape(C, HO, WO), b, out)
    return jnp.transpose(out, (0, 1, 3, 2))


# speedup vs baseline: 1.1116x; 1.1116x over previous
import functools

import jax
import jax.numpy as jnp
from jax import lax
from jax.experimental import pallas as pl
from jax.experimental.pallas import tpu as pltpu
from jax.experimental.pallas import tpu_sc as plsc

B, H, W, C = 4, 112, 112, 192
HW = H * W
HO, WO = 2 * H, 2 * W
NCORE, NSUB = 2, 16
NWORK = NCORE * NSUB
MPERW = C // NWORK
HB = 16
YB = 16
NHB = H // HB
NYB = HO // YB
INV_WC = 1.0 / (WO * C)
INV_C = 1.0 / C


def _group_body(x_ref, i_ref, vo_hbm, po_hbm, vscr, pscr, sem):
    i = pl.program_id(0)
    sel = jax.lax.rem(i, 2)

    @pl.when(i >= 2)
    def _():
        pltpu.make_async_copy(
            vscr.at[sel], vo_hbm.at[:, pl.ds((i - 2) * HB, HB), :], sem.at[sel]).wait()
        pltpu.make_async_copy(
            pscr.at[sel], po_hbm.at[:, pl.ds((i - 2) * HB, HB), :], sem.at[sel]).wait()

    v = x_ref[0]
    idx = i_ref[0]
    y0 = (idx.astype(jnp.float32) * INV_WC).astype(jnp.int32)
    r = idx - y0 * (WO * C)
    fix = (r >= WO * C).astype(jnp.int32) - (r < 0).astype(jnp.int32)
    y = y0 + fix
    r = r - fix * (WO * C)
    x0 = (r.astype(jnp.float32) * INV_C).astype(jnp.int32)
    r2 = r - x0 * C
    x = x0 + (r2 >= C).astype(jnp.int32) - (r2 < 0).astype(jnp.int32)
    packed = jnp.left_shift(y, 8) + x
    vscr[sel] = jnp.swapaxes(v, 0, 1)
    pscr[sel] = jnp.swapaxes(packed, 0, 1)
    pltpu.async_copy(vscr.at[sel], vo_hbm.at[:, pl.ds(i * HB, HB), :], sem.at[sel])
    pltpu.async_copy(pscr.at[sel], po_hbm.at[:, pl.ds(i * HB, HB), :], sem.at[sel])

    @pl.when(i == NHB - 1)
    def _():
        for s in (1 - sel, sel):
            off = i - 1 + (s == sel)
            pltpu.make_async_copy(
                vscr.at[s], vo_hbm.at[:, pl.ds(off * HB, HB), :], sem.at[s]).wait()
            pltpu.make_async_copy(
                pscr.at[s], po_hbm.at[:, pl.ds(off * HB, HB), :], sem.at[s]).wait()


def _group_channels(x, ii, b):
    return pl.pallas_call(
        _group_body,
        grid=(NHB,),
        in_specs=[
            pl.BlockSpec((1, HB, C, W), lambda i: (b, i, 0, 0)),
            pl.BlockSpec((1, HB, C, W), lambda i: (b, i, 0, 0)),
        ],
        out_specs=[
            pl.BlockSpec(memory_space=pl.ANY),
            pl.BlockSpec(memory_space=pl.ANY),
        ],
        out_shape=[
            jax.ShapeDtypeStruct((C, H, W), jnp.float32),
            jax.ShapeDtypeStruct((C, H, W), jnp.int32),
        ],
        scratch_shapes=[
            pltpu.VMEM((2, C, HB, W), jnp.float32),
            pltpu.VMEM((2, C, HB, W), jnp.int32),
            pltpu.SemaphoreType.DMA((2,)),
        ],
    )(x, ii)


def _relayout_body(t_hbm, o_ref, scr, sem):
    y = pl.program_id(0)
    sel = jax.lax.rem(y, 2)

    @pl.when(y == 0)
    def _():
        pltpu.async_copy(t_hbm.at[:, pl.ds(0, YB), :], scr.at[0], sem.at[0])

    @pl.when(y < NYB - 1)
    def _():
        pltpu.async_copy(
            t_hbm.at[:, pl.ds((y + 1) * YB, YB), :], scr.at[1 - sel], sem.at[1 - sel])

    pltpu.make_async_copy(
        t_hbm.at[:, pl.ds(y * YB, YB), :], scr.at[sel], sem.at[sel]).wait()
    o_ref[0] = jnp.swapaxes(scr[sel], 0, 1)


def _relayout_accum_body(prev_ref, t_hbm, o_ref, scr, sem):
    del prev_ref
    _relayout_body(t_hbm, o_ref, scr, sem)


def _relayout_out(out_t, b, prev):
    scratch = [
        pltpu.VMEM((2, C, YB, WO), jnp.float32),
        pltpu.SemaphoreType.DMA((2,)),
    ]
    if prev is None:
        return pl.pallas_call(
            _relayout_body,
            grid=(NYB,),
            in_specs=[pl.BlockSpec(memory_space=pl.ANY)],
            out_specs=pl.BlockSpec((1, YB, C, WO), lambda y: (b, y, 0, 0)),
            out_shape=jax.ShapeDtypeStruct((B, HO, C, WO), jnp.float32),
            scratch_shapes=scratch,
        )(out_t)
    return pl.pallas_call(
        _relayout_accum_body,
        grid=(NYB,),
        in_specs=[
            pl.BlockSpec(memory_space=pl.ANY),
            pl.BlockSpec(memory_space=pl.ANY),
        ],
        out_specs=pl.BlockSpec((1, YB, C, WO), lambda y: (b, y, 0, 0)),
        out_shape=jax.ShapeDtypeStruct((B, HO, C, WO), jnp.float32),
        input_output_aliases={0: 0},
        scratch_shapes=scratch,
    )(prev, out_t)


_mesh = plsc.VectorSubcoreMesh(core_axis_name="c", subcore_axis_name="s")


@functools.partial(
    pl.kernel,
    out_type=jax.ShapeDtypeStruct((C, HO, WO), jnp.float32),
    mesh=_mesh,
    scratch_types=[
        pltpu.VMEM((HO, WO), jnp.float32),
        pltpu.VMEM((2, H, W), jnp.float32),
        pltpu.VMEM((2, H, W), jnp.int32),
        pltpu.SemaphoreType.DMA((2,)),
    ],
    compiler_params=pltpu.CompilerParams(needs_layout_passes=False),
)
def _scatter(vals_hbm, p_hbm, out_hbm, canvas, vbuf, pbuf, lsem):
    w = lax.axis_index("s") * NCORE + lax.axis_index("c")
    zero16 = jnp.zeros((16,), jnp.float32)
    mask8 = jnp.full((16,), 255, jnp.int32)

    def _load(m, sel):
        c = w + NWORK * m
        pltpu.async_copy(vals_hbm.at[c], vbuf.at[sel], lsem.at[sel])
        pltpu.async_copy(p_hbm.at[c], pbuf.at[sel], lsem.at[sel])

    _load(jnp.int32(0), jnp.int32(0))

    def _zero(y, carry):
        def _zrow(j, carry2):
            canvas[y, pl.ds(j * 16, 16)] = zero16
            return carry2
        return lax.fori_loop(0, WO // 16, _zrow, carry)

    lax.fori_loop(0, HO, _zero, 0)

    def _plane(m, carry):
        sel = jnp.bitwise_and(m, 1)

        @pl.when(m < MPERW - 1)
        def _():
            _load(m + 1, 1 - sel)

        c = w + NWORK * m
        pltpu.make_async_copy(vals_hbm.at[c], vbuf.at[sel], lsem.at[sel]).wait()
        pltpu.make_async_copy(p_hbm.at[c], pbuf.at[sel], lsem.at[sel]).wait()

        def _scat(h, carry3):
            for u in range(W // 16):
                pv = pbuf[sel, h, pl.ds(u * 16, 16)]
                yv = jnp.right_shift(pv, 8)
                xv = jnp.bitwise_and(pv, mask8)
                valv = vbuf[sel, h, pl.ds(u * 16, 16)]
                plsc.addupdate_scatter(canvas, [yv, xv], valv)
            return carry3

        lax.fori_loop(0, H, _scat, 0)
        pltpu.sync_copy(canvas, out_hbm.at[c])

        def _clear(h, carry3):
            for u in range(W // 16):
                pv = pbuf[sel, h, pl.ds(u * 16, 16)]
                yv = jnp.right_shift(pv, 8)
                xv = jnp.bitwise_and(pv, mask8)
                plsc.store_scatter(canvas, [yv, xv], zero16)
            return carry3

        lax.fori_loop(0, H, _clear, 0)
        return carry

    lax.fori_loop(0, MPERW, _plane, 0)


def kernel(input, indices):
    x = jnp.transpose(input, (0, 1, 3, 2))
    ii = jnp.transpose(indices.astype(jnp.int32), (0, 1, 3, 2))
    out = None
    for b in range(B):
        vt, pt = _group_channels(x, ii, b)
        out_t = _scatter(vt, pt)
        out = _relayout_out(out_t, b, out)
    return jnp.transpose(out, (0, 1, 3, 2))


# speedup vs baseline: 1.9052x; 1.7138x over previous
import functools

import jax
import jax.numpy as jnp
from jax import lax
from jax.experimental import pallas as pl
from jax.experimental.pallas import tpu as pltpu
from jax.experimental.pallas import tpu_sc as plsc

B, H, W, C = 4, 112, 112, 192
HW = H * W
HO, WO = 2 * H, 2 * W
NCORE, NSUB = 2, 16
NWORK = NCORE * NSUB
MPERW = C // NWORK
HB = 16
YB = 16
NHB = H // HB
NYB = HO // YB
INV_WC = 1.0 / (WO * C)
INV_C = 1.0 / C


def _group_body(x_ref, i_ref, vo_hbm, po_hbm, vscr, pscr, sem):
    i = pl.program_id(0)
    sel = jax.lax.rem(i, 2)

    @pl.when(i >= 2)
    def _():
        pltpu.make_async_copy(
            vscr.at[sel], vo_hbm.at[:, pl.ds((i - 2) * HB, HB), :], sem.at[sel]).wait()
        pltpu.make_async_copy(
            pscr.at[sel], po_hbm.at[:, pl.ds((i - 2) * HB, HB), :], sem.at[sel]).wait()

    v = x_ref[0]
    idx = i_ref[0]
    y0 = (idx.astype(jnp.float32) * INV_WC).astype(jnp.int32)
    r = idx - y0 * (WO * C)
    fix = (r >= WO * C).astype(jnp.int32) - (r < 0).astype(jnp.int32)
    y = y0 + fix
    r = r - fix * (WO * C)
    x0 = (r.astype(jnp.float32) * INV_C).astype(jnp.int32)
    r2 = r - x0 * C
    x = x0 + (r2 >= C).astype(jnp.int32) - (r2 < 0).astype(jnp.int32)
    packed = jnp.left_shift(y, 8) + x
    vscr[sel] = jnp.swapaxes(v, 0, 1)
    pscr[sel] = jnp.swapaxes(packed, 0, 1)
    pltpu.async_copy(vscr.at[sel], vo_hbm.at[:, pl.ds(i * HB, HB), :], sem.at[sel])
    pltpu.async_copy(pscr.at[sel], po_hbm.at[:, pl.ds(i * HB, HB), :], sem.at[sel])

    @pl.when(i == NHB - 1)
    def _():
        for s in (1 - sel, sel):
            off = i - 1 + (s == sel)
            pltpu.make_async_copy(
                vscr.at[s], vo_hbm.at[:, pl.ds(off * HB, HB), :], sem.at[s]).wait()
            pltpu.make_async_copy(
                pscr.at[s], po_hbm.at[:, pl.ds(off * HB, HB), :], sem.at[s]).wait()


def _group_channels(x, ii, b):
    return pl.pallas_call(
        _group_body,
        grid=(NHB,),
        in_specs=[
            pl.BlockSpec((1, HB, C, W), lambda i: (b, i, 0, 0)),
            pl.BlockSpec((1, HB, C, W), lambda i: (b, i, 0, 0)),
        ],
        out_specs=[
            pl.BlockSpec(memory_space=pl.ANY),
            pl.BlockSpec(memory_space=pl.ANY),
        ],
        out_shape=[
            jax.ShapeDtypeStruct((C, H, W), jnp.float32),
            jax.ShapeDtypeStruct((C, H, W), jnp.int32),
        ],
        scratch_shapes=[
            pltpu.VMEM((2, C, HB, W), jnp.float32),
            pltpu.VMEM((2, C, HB, W), jnp.int32),
            pltpu.SemaphoreType.DMA((2,)),
        ],
    )(x, ii)


def _relayout_body(t_hbm, o_ref, scr, sem):
    y = pl.program_id(0)
    sel = jax.lax.rem(y, 2)

    @pl.when(y == 0)
    def _():
        pltpu.async_copy(t_hbm.at[:, pl.ds(0, YB), :], scr.at[0], sem.at[0])

    @pl.when(y < NYB - 1)
    def _():
        pltpu.async_copy(
            t_hbm.at[:, pl.ds((y + 1) * YB, YB), :], scr.at[1 - sel], sem.at[1 - sel])

    pltpu.make_async_copy(
        t_hbm.at[:, pl.ds(y * YB, YB), :], scr.at[sel], sem.at[sel]).wait()
    o_ref[0] = jnp.swapaxes(scr[sel], 0, 1)


def _relayout_accum_body(prev_ref, t_hbm, o_ref, scr, sem):
    del prev_ref
    _relayout_body(t_hbm, o_ref, scr, sem)


def _relayout_out(out_t, b, prev):
    scratch = [
        pltpu.VMEM((2, C, YB, WO), jnp.float32),
        pltpu.SemaphoreType.DMA((2,)),
    ]
    if prev is None:
        return pl.pallas_call(
            _relayout_body,
            grid=(NYB,),
            in_specs=[pl.BlockSpec(memory_space=pl.ANY)],
            out_specs=pl.BlockSpec((1, YB, C, WO), lambda y: (b, y, 0, 0)),
            out_shape=jax.ShapeDtypeStruct((B, HO, C, WO), jnp.float32),
            scratch_shapes=scratch,
        )(out_t)
    return pl.pallas_call(
        _relayout_accum_body,
        grid=(NYB,),
        in_specs=[
            pl.BlockSpec(memory_space=pl.ANY),
            pl.BlockSpec(memory_space=pl.ANY),
        ],
        out_specs=pl.BlockSpec((1, YB, C, WO), lambda y: (b, y, 0, 0)),
        out_shape=jax.ShapeDtypeStruct((B, HO, C, WO), jnp.float32),
        input_output_aliases={0: 0},
        scratch_shapes=scratch,
    )(prev, out_t)


_mesh = plsc.VectorSubcoreMesh(core_axis_name="c", subcore_axis_name="s")


@functools.partial(
    pl.kernel,
    out_type=jax.ShapeDtypeStruct((C, HO, WO), jnp.float32),
    mesh=_mesh,
    scratch_types=[
        pltpu.VMEM((HO, WO), jnp.float32),
        pltpu.VMEM((2, H, W), jnp.float32),
        pltpu.VMEM((2, H, W), jnp.int32),
        pltpu.SemaphoreType.DMA((2,)),
    ],
    compiler_params=pltpu.CompilerParams(needs_layout_passes=False),
)
def _scatter(vals_hbm, p_hbm, out_hbm, canvas, vbuf, pbuf, lsem):
    w = lax.axis_index("s") * NCORE + lax.axis_index("c")
    zero16 = jnp.zeros((16,), jnp.float32)
    mask8 = jnp.full((16,), 255, jnp.int32)

    def _load(m, sel):
        c = w + NWORK * m
        pltpu.async_copy(vals_hbm.at[c], vbuf.at[sel], lsem.at[sel])
        pltpu.async_copy(p_hbm.at[c], pbuf.at[sel], lsem.at[sel])

    _load(jnp.int32(0), jnp.int32(0))

    @plsc.parallel_loop(0, HO, unroll=2)
    def _zero(y):
        for j in range(WO // 16):
            canvas[y, pl.ds(j * 16, 16)] = zero16

    def _plane(m, carry):
        sel = jnp.bitwise_and(m, 1)

        @pl.when(m < MPERW - 1)
        def _():
            _load(m + 1, 1 - sel)

        c = w + NWORK * m
        pltpu.make_async_copy(vals_hbm.at[c], vbuf.at[sel], lsem.at[sel]).wait()
        pltpu.make_async_copy(p_hbm.at[c], pbuf.at[sel], lsem.at[sel]).wait()

        @plsc.parallel_loop(0, H, unroll=2)
        def _scat(h):
            for u in range(W // 16):
                pv = pbuf[sel, h, pl.ds(u * 16, 16)]
                yv = jnp.right_shift(pv, 8)
                xv = jnp.bitwise_and(pv, mask8)
                valv = vbuf[sel, h, pl.ds(u * 16, 16)]
                plsc.addupdate_scatter(canvas, [yv, xv], valv)
        pltpu.sync_copy(canvas, out_hbm.at[c])

        @plsc.parallel_loop(0, H, unroll=2)
        def _clear(h):
            for u in range(W // 16):
                pv = pbuf[sel, h, pl.ds(u * 16, 16)]
                yv = jnp.right_shift(pv, 8)
                xv = jnp.bitwise_and(pv, mask8)
                plsc.store_scatter(canvas, [yv, xv], zero16)
        return carry

    lax.fori_loop(0, MPERW, _plane, 0)


def kernel(input, indices):
    x = jnp.transpose(input, (0, 1, 3, 2))
    ii = jnp.transpose(indices.astype(jnp.int32), (0, 1, 3, 2))
    out = None
    for b in range(B):
        vt, pt = _group_channels(x, ii, b)
        out_t = _scatter(vt, pt)
        out = _relayout_out(out_t, b, out)
    return jnp.transpose(out, (0, 1, 3, 2))
